# Initial kernel scaffold; baseline (speedup 1.0000x reference)
import jax
import jax.numpy as jnp
from jax.experimental import pallas as pl


def kernel(input, edge_index, params):
    raise NotImplementedError("write your pallas kernel here")


# speedup vs baseline: 1.0011x; 1.0011x over previous
import jax
import jax.numpy as jnp
from jax.experimental import pallas as pl


def _head_kernel(h_ref, w0, b0, w1, b1, w2, b2, w3, b3, o_ref):
    h = h_ref[...]
    h = jnp.maximum(h @ w0[...] + b0[...], 0.0)
    h = jnp.maximum(h @ w1[...] + b1[...], 0.0)
    h = jnp.maximum(h @ w2[...] + b2[...], 0.0)
    o_ref[...] = h @ w3[...] + b3[...]


def _head(h, reg):
    n = h.shape[0]
    out_dim = reg[3]["W"].shape[1]
    grid = (n // 1000,)
    in_specs = [pl.BlockSpec((1000, h.shape[1]), lambda i: (i, 0))]
    for rp in reg:
        in_specs.append(pl.BlockSpec(rp["W"].shape, lambda i: (0, 0)))
        in_specs.append(pl.BlockSpec(rp["b"].shape, lambda i: (0,)))
    return pl.pallas_call(
        _head_kernel,
        grid=grid,
        in_specs=in_specs,
        out_specs=pl.BlockSpec((1000, out_dim), lambda i: (i, 0)),
        out_shape=jax.ShapeDtypeStruct((n, out_dim), jnp.float32),
    )(h, reg[0]["W"], reg[0]["b"], reg[1]["W"], reg[1]["b"],
      reg[2]["W"], reg[2]["b"], reg[3]["W"], reg[3]["b"])


def _norm(edge_index, num_nodes):
    row = edge_index[0]
    col = edge_index[1]
    ew = jnp.ones(row.shape[0], jnp.float32)
    ew = jnp.where(row != col, ew, 0.0)
    deg = jnp.zeros((num_nodes,), jnp.float32).at[row].add(ew)
    dis = jnp.where(deg > 0, deg ** -0.5, 0.0)
    w = -(dis[row] * ew * dis[col])
    return w


def kernel(input, edge_index, params):
    x = input
    N = x.shape[0]
    norm_w = _norm(edge_index, N)
    src = edge_index[0]
    dst = edge_index[1]

    def prop(h):
        return jnp.zeros((N, h.shape[1]), h.dtype).at[dst].add(norm_w[:, None] * h[src])

    def cheb(h, p):
        Ws, b = p["Ws"], p["b"]
        Tx0 = h
        out = Tx0 @ Ws[0]
        Tx1 = Tx0
        if len(Ws) > 1:
            Tx1 = prop(Tx0)
            out = out + Tx1 @ Ws[1]
        for W in Ws[2:]:
            Tx2 = 2.0 * prop(Tx1) - Tx0
            out = out + Tx2 @ W
            Tx0, Tx1 = Tx1, Tx2
        return out + b

    def fwd_block(bp, h, res):
        z = h
        for lp in bp:
            z = jax.nn.relu(cheb(z, lp))
        return jax.nn.relu(z + res)

    h = x
    for i, bp in enumerate(params["blocks"]):
        if i == 0:
            res = cheb(h, params["res_layers"][0])
        elif i == 3:
            res = cheb(h, params["res_layers"][1])
        elif i == 7:
            res = cheb(h, params["res_layers"][2])
        elif i == 13:
            res = cheb(h, params["res_layers"][3])
        else:
            res = h
        h = fwd_block(bp, h, res)
    return _head(h, params["reg"])


# speedup vs baseline: 6.6011x; 6.5940x over previous
import functools

import jax
import jax.numpy as jnp
from jax import lax
from jax.experimental import pallas as pl
from jax.experimental.pallas import tpu as pltpu
from jax.experimental.pallas import tpu_sc as plsc

N = 10000
PAD_ROWS = 64
NPAD = N + PAD_ROWS
E_RAW = 320000
W_E = 128
N_WIN = 158
EPT = W_E * N_WIN
E_PAD = EPT * 16
ROWS_PT = 624


@functools.lru_cache(maxsize=None)
def _make_spmv(F2):
    mesh = plsc.VectorSubcoreMesh(core_axis_name="c", subcore_axis_name="s")

    @functools.partial(
        pl.kernel,
        mesh=mesh,
        compiler_params=pltpu.CompilerParams(use_tc_tiling_on_sc=False),
        out_type=(jax.ShapeDtypeStruct((N, F2), jnp.float32),
                  jax.ShapeDtypeStruct((N, F2), jnp.float32)),
        scratch_types=[
            pltpu.VMEM((N_WIN, W_E), jnp.int32),
            pltpu.VMEM((N_WIN, W_E), jnp.int32),
            pltpu.VMEM((W_E, F2), jnp.float32),
            pltpu.VMEM_SHARED((N, F2), jnp.float32),
            pltpu.SemaphoreType.DMA,
        ],
    )
    def spmv(hs0, hs1, zeros_hbm, src_hbm, dst_hbm, y0, y1,
             src_v, dst_v, gbuf, acc, sem):
        cid = lax.axis_index("c")
        sid = lax.axis_index("s")
        r0 = sid * ROWS_PT

        pltpu.sync_copy(zeros_hbm, acc.at[pl.ds(r0, ROWS_PT)])

        @pl.when(sid == 15)
        def _():
            pltpu.sync_copy(zeros_hbm.at[pl.ds(0, 16)],
                            acc.at[pl.ds(16 * ROWS_PT, 16)])

        pltpu.sync_copy(src_hbm.at[sid], src_v)
        pltpu.sync_copy(dst_hbm.at[sid], dst_v)
        plsc.subcore_barrier()

        for core_k, hs in ((0, hs0), (1, hs1)):
            @pl.when(cid == core_k)
            def _(hs=hs):
                def win(w, carry):
                    pltpu.async_copy(hs.at[src_v.at[w]], gbuf, sem).wait()
                    pltpu.sync_copy(gbuf, acc.at[dst_v.at[w]], add=True)
                    return carry

                lax.fori_loop(0, N_WIN, win, 0)

        plsc.subcore_barrier()

        for core_k, yout in ((0, y0), (1, y1)):
            @pl.when(cid == core_k)
            def _(yout=yout):
                pltpu.sync_copy(acc.at[pl.ds(r0, ROWS_PT)],
                                yout.at[pl.ds(r0, ROWS_PT)])

                @pl.when(sid == 15)
                def _():
                    pltpu.sync_copy(acc.at[pl.ds(16 * ROWS_PT, 16)],
                                    yout.at[pl.ds(16 * ROWS_PT, 16)])

    return spmv


def _pad_tbl(hs):
    return jnp.concatenate(
        [hs, jnp.zeros((PAD_ROWS, hs.shape[1]), jnp.float32)], axis=0)


def _head_kernel(h_ref, w0, b0, w1, b1, w2, b2, w3, b3, o_ref):
    h = h_ref[...]
    h = jnp.maximum(h @ w0[...] + b0[...], 0.0)
    h = jnp.maximum(h @ w1[...] + b1[...], 0.0)
    h = jnp.maximum(h @ w2[...] + b2[...], 0.0)
    o_ref[...] = h @ w3[...] + b3[...]


def _head(h, reg):
    n = h.shape[0]
    out_dim = reg[3]["W"].shape[1]
    grid = (n // 1000,)
    in_specs = [pl.BlockSpec((1000, h.shape[1]), lambda i: (i, 0))]
    for rp in reg:
        in_specs.append(pl.BlockSpec(rp["W"].shape, lambda i: (0, 0)))
        in_specs.append(pl.BlockSpec(rp["b"].shape, lambda i: (0,)))
    return pl.pallas_call(
        _head_kernel,
        grid=grid,
        in_specs=in_specs,
        out_specs=pl.BlockSpec((1000, out_dim), lambda i: (i, 0)),
        out_shape=jax.ShapeDtypeStruct((n, out_dim), jnp.float32),
    )(h, reg[0]["W"], reg[0]["b"], reg[1]["W"], reg[1]["b"],
      reg[2]["W"], reg[2]["b"], reg[3]["W"], reg[3]["b"])


def kernel(input, edge_index, params):
    x = input
    src = edge_index[0]
    dst = edge_index[1]
    self_loop = src == dst
    eidx = jnp.arange(E_RAW)
    tail = N + (eidx % PAD_ROWS)
    pad_n = E_PAD - E_RAW
    pidx = jnp.arange(pad_n)
    pad_tail = N + (pidx % PAD_ROWS)
    pad_spread = (pidx * 997) % N

    def mk_edges(g, s):
        g3 = jnp.concatenate([g, pad_tail]).astype(jnp.int32)
        s3 = jnp.concatenate([s, pad_spread]).astype(jnp.int32)
        return g3.reshape(16, N_WIN, W_E), s3.reshape(16, N_WIN, W_E)

    ones_tbl = _pad_tbl(jnp.ones((N, 8), jnp.float32))
    z8 = jnp.zeros((ROWS_PT, 8), jnp.float32)
    dg, ds_ = mk_edges(jnp.where(self_loop, tail, dst), src)
    d0, _ = _make_spmv(8)(ones_tbl, ones_tbl, z8, dg, ds_)
    deg = d0[:, 0]
    dis = jnp.where(deg > 0, lax.rsqrt(deg), 0.0)

    pg, psc = mk_edges(jnp.where(self_loop, tail, src), dst)

    def prop(h):
        F2 = h.shape[1] // 2
        hs = _pad_tbl(dis[:, None] * h)
        zf = jnp.zeros((ROWS_PT, F2), jnp.float32)
        y0, y1 = _make_spmv(F2)(hs[:, :F2], hs[:, F2:], zf, pg, psc)
        return (-dis)[:, None] * jnp.concatenate([y0, y1], axis=1)

    def cheb(h, p):
        Ws, b = p["Ws"], p["b"]
        Tx0 = h
        out = Tx0 @ Ws[0]
        Tx1 = Tx0
        if len(Ws) > 1:
            Tx1 = prop(Tx0)
            out = out + Tx1 @ Ws[1]
        for W in Ws[2:]:
            Tx2 = 2.0 * prop(Tx1) - Tx0
            out = out + Tx2 @ W
            Tx0, Tx1 = Tx1, Tx2
        return out + b

    def fwd_block(bp, h, res):
        z = h
        for lp in bp:
            z = jax.nn.relu(cheb(z, lp))
        return jax.nn.relu(z + res)

    h = x
    for i, bp in enumerate(params["blocks"]):
        if i == 0:
            res = cheb(h, params["res_layers"][0])
        elif i == 3:
            res = cheb(h, params["res_layers"][1])
        elif i == 7:
            res = cheb(h, params["res_layers"][2])
        elif i == 13:
            res = cheb(h, params["res_layers"][3])
        else:
            res = h
        h = fwd_block(bp, h, res)
    return _head(h, params["reg"])


# speedup vs baseline: 12.7651x; 1.9338x over previous
import functools

import jax
import jax.numpy as jnp
from jax import lax
from jax.experimental import pallas as pl
from jax.experimental.pallas import tpu as pltpu
from jax.experimental.pallas import tpu_sc as plsc

N = 10000
PAD_ROWS = 64
NPAD = N + PAD_ROWS
E_RAW = 320000
W_E = 128
N_WIN = 160
EPT = W_E * N_WIN
E_PAD = EPT * 16
NBUF = 4
ROWS_PT = 624


@functools.lru_cache(maxsize=None)
def _make_spmv(F2):
    mesh = plsc.VectorSubcoreMesh(core_axis_name="c", subcore_axis_name="s")

    @functools.partial(
        pl.kernel,
        mesh=mesh,
        compiler_params=pltpu.CompilerParams(use_tc_tiling_on_sc=False),
        out_type=(jax.ShapeDtypeStruct((N, F2), jnp.float32),
                  jax.ShapeDtypeStruct((N, F2), jnp.float32)),
        scratch_types=[
            pltpu.VMEM((N_WIN, W_E), jnp.int32),
            pltpu.VMEM((N_WIN, W_E), jnp.int32),
            pltpu.VMEM((NBUF, W_E, F2), jnp.float32),
            pltpu.VMEM_SHARED((N, F2), jnp.float32),
            [pltpu.SemaphoreType.DMA] * NBUF,
            [pltpu.SemaphoreType.DMA] * NBUF,
        ],
    )
    def spmv(hs0, hs1, zeros_hbm, src_hbm, dst_hbm, y0, y1,
             src_v, dst_v, gbuf, acc, gsem, ssem):
        cid = lax.axis_index("c")
        sid = lax.axis_index("s")
        r0 = sid * ROWS_PT

        pltpu.sync_copy(zeros_hbm, acc.at[pl.ds(r0, ROWS_PT)])

        @pl.when(sid == 15)
        def _():
            pltpu.sync_copy(zeros_hbm.at[pl.ds(0, 16)],
                            acc.at[pl.ds(16 * ROWS_PT, 16)])

        pltpu.sync_copy(src_hbm.at[sid], src_v)
        pltpu.sync_copy(dst_hbm.at[sid], dst_v)
        plsc.subcore_barrier()

        for core_k, hs in ((0, hs0), (1, hs1)):
            @pl.when(cid == core_k)
            def _(hs=hs):
                def g_cp(w, b):
                    return pltpu.make_async_copy(
                        hs.at[src_v.at[w]], gbuf.at[b], gsem[b])

                def s_cp(w, b):
                    return pltpu.make_async_copy(
                        gbuf.at[b], acc.at[dst_v.at[w]], ssem[b])

                for b in range(NBUF - 1):
                    g_cp(b, b).start()

                def body(k, carry):
                    for b in range(NBUF):
                        w = NBUF * k + b
                        g_cp(w, b).wait()
                        s_cp(w, b).start(add=True)
                        if b == 0:
                            @pl.when(k >= 1)
                            def _():
                                s_cp(w - 1, NBUF - 1).wait()
                            g_cp(w + NBUF - 1, NBUF - 1).start()
                        else:
                            s_cp(w - 1, b - 1).wait()
                            if b == 1:
                                @pl.when(k < N_WIN // NBUF - 1)
                                def _():
                                    g_cp(w + NBUF - 1, 0).start()
                            else:
                                @pl.when(k < N_WIN // NBUF - 1)
                                def _():
                                    g_cp(w + NBUF - 1, b - 1).start()
                    return carry

                lax.fori_loop(0, N_WIN // NBUF, body, 0)
                s_cp(N_WIN - 1, NBUF - 1).wait()

        plsc.subcore_barrier()

        for core_k, yout in ((0, y0), (1, y1)):
            @pl.when(cid == core_k)
            def _(yout=yout):
                pltpu.sync_copy(acc.at[pl.ds(r0, ROWS_PT)],
                                yout.at[pl.ds(r0, ROWS_PT)])

                @pl.when(sid == 15)
                def _():
                    pltpu.sync_copy(acc.at[pl.ds(16 * ROWS_PT, 16)],
                                    yout.at[pl.ds(16 * ROWS_PT, 16)])

    return spmv


def _pad_tbl(hs):
    return jnp.concatenate(
        [hs, jnp.zeros((PAD_ROWS, hs.shape[1]), jnp.float32)], axis=0)


def _head_kernel(h_ref, w0, b0, w1, b1, w2, b2, w3, b3, o_ref):
    h = h_ref[...]
    h = jnp.maximum(h @ w0[...] + b0[...], 0.0)
    h = jnp.maximum(h @ w1[...] + b1[...], 0.0)
    h = jnp.maximum(h @ w2[...] + b2[...], 0.0)
    o_ref[...] = h @ w3[...] + b3[...]


def _head(h, reg):
    n = h.shape[0]
    out_dim = reg[3]["W"].shape[1]
    grid = (n // 1000,)
    in_specs = [pl.BlockSpec((1000, h.shape[1]), lambda i: (i, 0))]
    for rp in reg:
        in_specs.append(pl.BlockSpec(rp["W"].shape, lambda i: (0, 0)))
        in_specs.append(pl.BlockSpec(rp["b"].shape, lambda i: (0,)))
    return pl.pallas_call(
        _head_kernel,
        grid=grid,
        in_specs=in_specs,
        out_specs=pl.BlockSpec((1000, out_dim), lambda i: (i, 0)),
        out_shape=jax.ShapeDtypeStruct((n, out_dim), jnp.float32),
    )(h, reg[0]["W"], reg[0]["b"], reg[1]["W"], reg[1]["b"],
      reg[2]["W"], reg[2]["b"], reg[3]["W"], reg[3]["b"])


def kernel(input, edge_index, params):
    x = input
    src = edge_index[0]
    dst = edge_index[1]
    self_loop = src == dst
    eidx = jnp.arange(E_RAW)
    tail = N + (eidx % PAD_ROWS)
    pad_n = E_PAD - E_RAW
    pidx = jnp.arange(pad_n)
    pad_tail = N + (pidx % PAD_ROWS)
    pad_spread = (pidx * 997) % N

    def mk_edges(g, s):
        g3 = jnp.concatenate([g, pad_tail]).astype(jnp.int32)
        s3 = jnp.concatenate([s, pad_spread]).astype(jnp.int32)
        return g3.reshape(16, N_WIN, W_E), s3.reshape(16, N_WIN, W_E)

    ones_tbl = _pad_tbl(jnp.ones((N, 8), jnp.float32))
    z8 = jnp.zeros((ROWS_PT, 8), jnp.float32)
    dg, ds_ = mk_edges(jnp.where(self_loop, tail, dst), src)
    d0, _ = _make_spmv(8)(ones_tbl, ones_tbl, z8, dg, ds_)
    deg = d0[:, 0]
    dis = jnp.where(deg > 0, lax.rsqrt(deg), 0.0)

    pg, psc = mk_edges(jnp.where(self_loop, tail, src), dst)

    def prop(h):
        F2 = h.shape[1] // 2
        hs = _pad_tbl(dis[:, None] * h)
        zf = jnp.zeros((ROWS_PT, F2), jnp.float32)
        y0, y1 = _make_spmv(F2)(hs[:, :F2], hs[:, F2:], zf, pg, psc)
        return (-dis)[:, None] * jnp.concatenate([y0, y1], axis=1)

    def cheb(h, p):
        Ws, b = p["Ws"], p["b"]
        Tx0 = h
        out = Tx0 @ Ws[0]
        Tx1 = Tx0
        if len(Ws) > 1:
            Tx1 = prop(Tx0)
            out = out + Tx1 @ Ws[1]
        for W in Ws[2:]:
            Tx2 = 2.0 * prop(Tx1) - Tx0
            out = out + Tx2 @ W
            Tx0, Tx1 = Tx1, Tx2
        return out + b

    def fwd_block(bp, h, res):
        z = h
        for lp in bp:
            z = jax.nn.relu(cheb(z, lp))
        return jax.nn.relu(z + res)

    h = x
    for i, bp in enumerate(params["blocks"]):
        if i == 0:
            res = cheb(h, params["res_layers"][0])
        elif i == 3:
            res = cheb(h, params["res_layers"][1])
        elif i == 7:
            res = cheb(h, params["res_layers"][2])
        elif i == 13:
            res = cheb(h, params["res_layers"][3])
        else:
            res = h
        h = fwd_block(bp, h, res)
    return _head(h, params["reg"])
